# baseline (speedup 1.0000x reference)
import functools

import jax
import jax.numpy as jnp
from jax import lax
from jax.experimental import pallas as pl
from jax.experimental.pallas import tpu as pltpu
from jax.experimental.pallas import tpu_sc as plsc

_LANES = 16


def _tec_kernel(n_pts, n_workers, chunk, n_nodes, n_cells,
                x_hbm, cid_hbm, coord_hbm, conn0_hbm, conn1_hbm, out_hbm,
                coord_v, conn0_v, conn1_v, x_v, cid_v, out_v):
    per_worker = n_pts // n_workers
    n_chunks = per_worker // chunk
    wid = lax.axis_index("s") * 2 + lax.axis_index("c")
    base = wid * per_worker

    pltpu.sync_copy(coord_hbm, coord_v)
    pltpu.sync_copy(conn0_hbm, conn0_v)
    pltpu.sync_copy(conn1_hbm, conn1_v)

    lane = lax.iota(jnp.int32, _LANES)

    def chunk_body(j, _):
        off = base + j * chunk
        pltpu.sync_copy(x_hbm.at[pl.ds(off, chunk)], x_v)
        pltpu.sync_copy(cid_hbm.at[pl.ds(off, chunk)], cid_v)

        def vec_body(k, _):
            o = k * _LANES
            cid = cid_v[pl.ds(o, _LANES)]
            n0 = plsc.load_gather(conn0_v, [cid])
            n1 = plsc.load_gather(conn1_v, [cid])
            x0 = plsc.load_gather(coord_v, [n0])
            x1 = plsc.load_gather(coord_v, [n1])
            xv = x_v[pl.ds(o, _LANES)]
            inv = 1.0 / (x0 - x1)
            na = (xv - x1) * inv
            nb = (x0 - xv) * inv
            pos = (o + lane) * 2
            plsc.store_scatter(out_v, [pos], na)
            plsc.store_scatter(out_v, [pos + 1], nb)
            return _

        lax.fori_loop(0, chunk // _LANES, vec_body, None, unroll=4)
        pltpu.sync_copy(out_v, out_hbm.at[pl.ds(2 * off, 2 * chunk)])
        return _

    lax.fori_loop(0, n_chunks, chunk_body, None)


def kernel(x, cell_id, coordinates, connectivity):
    n_pts = x.shape[0]
    n_nodes = coordinates.shape[0]
    n_cells = connectivity.shape[0]
    n_workers = 32
    chunk = 8192

    coord_flat = coordinates[:, 0]
    conn0 = connectivity[:, 0]
    conn1 = connectivity[:, 1]

    mesh = plsc.VectorSubcoreMesh(core_axis_name="c", subcore_axis_name="s")
    body = functools.partial(_tec_kernel, n_pts, n_workers, chunk,
                             n_nodes, n_cells)
    out_flat = pl.kernel(
        body,
        mesh=mesh,
        out_type=jax.ShapeDtypeStruct((2 * n_pts,), jnp.float32),
        compiler_params=pltpu.CompilerParams(
            needs_layout_passes=False, use_tc_tiling_on_sc=True),
        scratch_types=[
            pltpu.VMEM((n_nodes,), jnp.float32),
            pltpu.VMEM((n_cells,), jnp.int32),
            pltpu.VMEM((n_cells,), jnp.int32),
            pltpu.VMEM((chunk,), jnp.float32),
            pltpu.VMEM((chunk,), jnp.int32),
            pltpu.VMEM((2 * chunk,), jnp.float32),
        ],
    )(x, cell_id, coord_flat, conn0, conn1)
    return out_flat.reshape(n_pts, 2)


# speedup vs baseline: 11.0547x; 11.0547x over previous
import functools

import jax
import jax.numpy as jnp
from jax import lax
from jax.experimental import pallas as pl
from jax.experimental.pallas import tpu as pltpu
from jax.experimental.pallas import tpu_sc as plsc

_LANES = 16


def _tec_kernel(n_pts, n_workers, chunk, n_nodes, n_cells,
                x_hbm, cid_hbm, coord_hbm, conn0_hbm, conn1_hbm, out_hbm,
                coord_v, conn0_v, conn1_v, x_v, cid_v, out_v):
    per_worker = n_pts // n_workers
    n_chunks = per_worker // chunk
    wid = lax.axis_index("s") * 2 + lax.axis_index("c")
    base = wid * per_worker

    pltpu.sync_copy(coord_hbm, coord_v)
    pltpu.sync_copy(conn0_hbm, conn0_v)
    pltpu.sync_copy(conn1_hbm, conn1_v)

    lane = lax.iota(jnp.int32, _LANES)
    zeros = jnp.zeros((_LANES,), jnp.int32)
    ones = jnp.ones((_LANES,), jnp.int32)

    def chunk_body(j, _):
        off = base + j * chunk
        pltpu.sync_copy(x_hbm.at[pl.ds(off, chunk)], x_v)
        pltpu.sync_copy(cid_hbm.at[pl.ds(off, chunk)], cid_v)

        def vec_body(k, _):
            o = k * _LANES
            cid = cid_v[pl.ds(o, _LANES)]
            n0 = plsc.load_gather(conn0_v, [cid])
            n1 = plsc.load_gather(conn1_v, [cid])
            x0 = plsc.load_gather(coord_v, [n0])
            x1 = plsc.load_gather(coord_v, [n1])
            xv = x_v[pl.ds(o, _LANES)]
            inv = 1.0 / (x0 - x1)
            na = (xv - x1) * inv
            nb = (x0 - xv) * inv
            pos = o + (o // 128) * 128 + lane
            plsc.store_scatter(out_v, [pos], na)
            plsc.store_scatter(out_v, [pos + 128], nb)
            return _

        lax.fori_loop(0, chunk // _LANES, vec_body, None, unroll=4)
        pltpu.sync_copy(out_v, out_hbm.at[pl.ds(2 * off, 2 * chunk)])
        return _

    lax.fori_loop(0, n_chunks, chunk_body, None)


def kernel(x, cell_id, coordinates, connectivity):
    n_pts = x.shape[0]
    n_nodes = coordinates.shape[0]
    n_cells = connectivity.shape[0]
    n_workers = 32
    chunk = 8192

    coord_flat = coordinates[:, 0]
    conn0 = connectivity[:, 0]
    conn1 = connectivity[:, 1]

    mesh = plsc.VectorSubcoreMesh(core_axis_name="c", subcore_axis_name="s")
    body = functools.partial(_tec_kernel, n_pts, n_workers, chunk,
                             n_nodes, n_cells)
    out_flat = pl.kernel(
        body,
        mesh=mesh,
        out_type=jax.ShapeDtypeStruct((2 * n_pts,), jnp.float32),
        compiler_params=pltpu.CompilerParams(needs_layout_passes=False),
        scratch_types=[
            pltpu.VMEM((n_nodes,), jnp.float32),
            pltpu.VMEM((n_cells,), jnp.int32),
            pltpu.VMEM((n_cells,), jnp.int32),
            pltpu.VMEM((chunk,), jnp.float32),
            pltpu.VMEM((chunk,), jnp.int32),
            pltpu.VMEM((2 * chunk,), jnp.float32),
        ],
    )(x, cell_id, coord_flat, conn0, conn1)
    return out_flat.reshape(n_pts // 128, 2, 128).transpose(0, 2, 1).reshape(
        n_pts, 2)


# speedup vs baseline: 34.4878x; 3.1197x over previous
import functools

import jax
import jax.numpy as jnp
from jax import lax
from jax.experimental import pallas as pl
from jax.experimental.pallas import tpu as pltpu
from jax.experimental.pallas import tpu_sc as plsc

_LANES = 16


def _tec_kernel(n_pts, n_workers, chunk, n_cells,
                x_hbm, cid_hbm, coord_hbm, conn0_hbm, conn1_hbm, out_hbm,
                coord_v, conn0_v, conn1_v, x1t_v, invt_v,
                xb0, xb1, cb0, cb1, ob0, ob1,
                sx0, sx1, sc0, sc1, so0, so1):
    per_worker = n_pts // n_workers
    n_chunks = per_worker // chunk
    wid = lax.axis_index("s") * 2 + lax.axis_index("c")
    base = wid * per_worker

    xb = (xb0, xb1)
    cb = (cb0, cb1)
    ob = (ob0, ob1)
    sx = (sx0, sx1)
    sc = (sc0, sc1)
    so = (so0, so1)

    pltpu.sync_copy(coord_hbm, coord_v)
    pltpu.sync_copy(conn0_hbm, conn0_v)
    pltpu.sync_copy(conn1_hbm, conn1_v)
    for t in range(n_cells // _LANES):
        ds = pl.ds(t * _LANES, _LANES)
        n0 = conn0_v[ds]
        n1 = conn1_v[ds]
        x0 = plsc.load_gather(coord_v, [n0])
        x1 = plsc.load_gather(coord_v, [n1])
        x1t_v[ds] = x1
        invt_v[ds] = 1.0 / (x0 - x1)

    def start_in(j):
        off = base + j * chunk
        bi = j % 2
        hx = pltpu.async_copy(x_hbm.at[pl.ds(off, chunk)], xb[bi], sx[bi])
        hc = pltpu.async_copy(cid_hbm.at[pl.ds(off, chunk)], cb[bi], sc[bi])
        return hx, hc

    def start_out(j):
        off = base + j * chunk
        bi = j % 2
        return pltpu.async_copy(ob[bi], out_hbm.at[pl.ds(2 * off, 2 * chunk)],
                                so[bi])

    def compute(bi):
        x_v, cid_v, out_v = xb[bi], cb[bi], ob[bi]

        @plsc.parallel_loop(0, chunk // 128, unroll=2)
        def blk(bk):
            for s in range(128 // _LANES):
                o = bk * 128 + s * _LANES
                cid = cid_v[pl.ds(o, _LANES)]
                x1 = plsc.load_gather(x1t_v, [cid])
                inv = plsc.load_gather(invt_v, [cid])
                xv = x_v[pl.ds(o, _LANES)]
                na = (xv - x1) * inv
                p = bk * 256 + s * _LANES
                out_v[pl.ds(p, _LANES)] = na
                out_v[pl.ds(p + 128, _LANES)] = 1.0 - na

    h_in = [None] * n_chunks
    h_out = [None] * n_chunks
    h_in[0] = start_in(0)
    if n_chunks > 1:
        h_in[1] = start_in(1)
    for j in range(n_chunks):
        h_in[j][0].wait()
        h_in[j][1].wait()
        if j >= 2:
            h_out[j - 2].wait()
        compute(j % 2)
        h_out[j] = start_out(j)
        if j + 2 < n_chunks:
            h_in[j + 2] = start_in(j + 2)
    if n_chunks > 1:
        h_out[n_chunks - 2].wait()
    h_out[n_chunks - 1].wait()


def kernel(x, cell_id, coordinates, connectivity):
    n_pts = x.shape[0]
    n_nodes = coordinates.shape[0]
    n_cells = connectivity.shape[0]
    n_workers = 32
    chunk = 8192

    coord_flat = coordinates[:, 0]
    conn0 = connectivity[:, 0]
    conn1 = connectivity[:, 1]

    mesh = plsc.VectorSubcoreMesh(core_axis_name="c", subcore_axis_name="s")
    body = functools.partial(_tec_kernel, n_pts, n_workers, chunk, n_cells)
    out_flat = pl.kernel(
        body,
        mesh=mesh,
        out_type=jax.ShapeDtypeStruct((2 * n_pts,), jnp.float32),
        compiler_params=pltpu.CompilerParams(needs_layout_passes=False),
        scratch_types=[
            pltpu.VMEM((n_nodes,), jnp.float32),
            pltpu.VMEM((n_cells,), jnp.int32),
            pltpu.VMEM((n_cells,), jnp.int32),
            pltpu.VMEM((n_cells,), jnp.float32),
            pltpu.VMEM((n_cells,), jnp.float32),
            pltpu.VMEM((chunk,), jnp.float32),
            pltpu.VMEM((chunk,), jnp.float32),
            pltpu.VMEM((chunk,), jnp.int32),
            pltpu.VMEM((chunk,), jnp.int32),
            pltpu.VMEM((2 * chunk,), jnp.float32),
            pltpu.VMEM((2 * chunk,), jnp.float32),
            pltpu.SemaphoreType.DMA,
            pltpu.SemaphoreType.DMA,
            pltpu.SemaphoreType.DMA,
            pltpu.SemaphoreType.DMA,
            pltpu.SemaphoreType.DMA,
            pltpu.SemaphoreType.DMA,
        ],
    )(x, cell_id, coord_flat, conn0, conn1)
    return out_flat.reshape(n_pts // 128, 2, 128).transpose(0, 2, 1).reshape(
        n_pts, 2)


# speedup vs baseline: 35.8278x; 1.0389x over previous
import functools

import jax
import jax.numpy as jnp
from jax import lax
from jax.experimental import pallas as pl
from jax.experimental.pallas import tpu as pltpu
from jax.experimental.pallas import tpu_sc as plsc

_LANES = 16


def _tec_kernel(n_pts, n_workers, chunk, n_cells,
                x_hbm, cid_hbm, coord_hbm, conn0_hbm, conn1_hbm, out_hbm,
                coord_v, conn0_v, conn1_v, x1t_v, invt_v,
                xb0, xb1, cb0, cb1, ob0, ob1,
                sx0, sx1, sc0, sc1, so0, so1, st0):
    per_worker = n_pts // n_workers
    n_chunks = per_worker // chunk
    wid = lax.axis_index("s") * 2 + lax.axis_index("c")
    base = wid * per_worker

    xb = (xb0, xb1)
    cb = (cb0, cb1)
    ob = (ob0, ob1)
    sx = (sx0, sx1)
    sc = (sc0, sc1)
    so = (so0, so1)

    def start_in(j):
        off = base + j * chunk
        bi = j % 2
        hx = pltpu.async_copy(x_hbm.at[pl.ds(off, chunk)], xb[bi], sx[bi])
        hc = pltpu.async_copy(cid_hbm.at[pl.ds(off, chunk)], cb[bi], sc[bi])
        return hx, hc

    def start_out(j):
        off = base + j * chunk
        bi = j % 2
        return pltpu.async_copy(ob[bi], out_hbm.at[pl.ds(2 * off, 2 * chunk)],
                                so[bi])

    def compute(bi):
        x_v, cid_v, out_v = xb[bi], cb[bi], ob[bi]

        @plsc.parallel_loop(0, chunk // 128, unroll=4)
        def blk(bk):
            for s in range(128 // _LANES):
                o = bk * 128 + s * _LANES
                cid = cid_v[pl.ds(o, _LANES)]
                x1 = plsc.load_gather(x1t_v, [cid])
                inv = plsc.load_gather(invt_v, [cid])
                xv = x_v[pl.ds(o, _LANES)]
                na = (xv - x1) * inv
                p = bk * 256 + s * _LANES
                out_v[pl.ds(p, _LANES)] = na
                out_v[pl.ds(p + 128, _LANES)] = 1.0 - na

    ht0 = pltpu.async_copy(coord_hbm, coord_v, so0)
    ht1 = pltpu.async_copy(conn0_hbm, conn0_v, so1)
    ht2 = pltpu.async_copy(conn1_hbm, conn1_v, st0)

    h_in = [None] * n_chunks
    h_out = [None] * n_chunks
    h_in[0] = start_in(0)
    if n_chunks > 1:
        h_in[1] = start_in(1)

    ht0.wait()
    ht1.wait()
    ht2.wait()
    for t in range(n_cells // _LANES):
        ds = pl.ds(t * _LANES, _LANES)
        n0 = conn0_v[ds]
        n1 = conn1_v[ds]
        x0 = plsc.load_gather(coord_v, [n0])
        x1 = plsc.load_gather(coord_v, [n1])
        x1t_v[ds] = x1
        invt_v[ds] = 1.0 / (x0 - x1)

    for j in range(n_chunks):
        h_in[j][0].wait()
        h_in[j][1].wait()
        if j >= 2:
            h_out[j - 2].wait()
        compute(j % 2)
        h_out[j] = start_out(j)
        if j + 2 < n_chunks:
            h_in[j + 2] = start_in(j + 2)
    if n_chunks > 1:
        h_out[n_chunks - 2].wait()
    h_out[n_chunks - 1].wait()


def kernel(x, cell_id, coordinates, connectivity):
    n_pts = x.shape[0]
    n_nodes = coordinates.shape[0]
    n_cells = connectivity.shape[0]
    n_workers = 32
    chunk = 8192

    coord_flat = coordinates[:, 0]
    conn0 = connectivity[:, 0]
    conn1 = connectivity[:, 1]

    mesh = plsc.VectorSubcoreMesh(core_axis_name="c", subcore_axis_name="s")
    body = functools.partial(_tec_kernel, n_pts, n_workers, chunk, n_cells)
    out_flat = pl.kernel(
        body,
        mesh=mesh,
        out_type=jax.ShapeDtypeStruct((2 * n_pts,), jnp.float32),
        compiler_params=pltpu.CompilerParams(needs_layout_passes=False),
        scratch_types=[
            pltpu.VMEM((n_nodes,), jnp.float32),
            pltpu.VMEM((n_cells,), jnp.int32),
            pltpu.VMEM((n_cells,), jnp.int32),
            pltpu.VMEM((n_cells,), jnp.float32),
            pltpu.VMEM((n_cells,), jnp.float32),
            pltpu.VMEM((chunk,), jnp.float32),
            pltpu.VMEM((chunk,), jnp.float32),
            pltpu.VMEM((chunk,), jnp.int32),
            pltpu.VMEM((chunk,), jnp.int32),
            pltpu.VMEM((2 * chunk,), jnp.float32),
            pltpu.VMEM((2 * chunk,), jnp.float32),
            pltpu.SemaphoreType.DMA,
            pltpu.SemaphoreType.DMA,
            pltpu.SemaphoreType.DMA,
            pltpu.SemaphoreType.DMA,
            pltpu.SemaphoreType.DMA,
            pltpu.SemaphoreType.DMA,
            pltpu.SemaphoreType.DMA,
        ],
    )(x, cell_id, coord_flat, conn0, conn1)
    return out_flat.reshape(n_pts // 128, 2, 128).transpose(0, 2, 1).reshape(
        n_pts, 2)


# speedup vs baseline: 41.8275x; 1.1675x over previous
import functools

import jax
import jax.numpy as jnp
from jax import lax
from jax.experimental import pallas as pl
from jax.experimental.pallas import tpu as pltpu
from jax.experimental.pallas import tpu_sc as plsc

_LANES = 16


def _tec_kernel(n_pts, n_workers, chunk, n_cells,
                x_hbm, cid_hbm, coord_hbm, conn0_hbm, conn1_hbm, out_hbm,
                coord_v, conn0_v, conn1_v, x1t_v, invt_v,
                xb0, xb1, cb0, cb1, ob0, ob1,
                sx0, sx1, sc0, sc1, so0, so1, st0):
    per_worker = n_pts // n_workers
    n_chunks = per_worker // chunk
    wid = lax.axis_index("s") * 2 + lax.axis_index("c")
    base = wid * per_worker

    xb = (xb0, xb1)
    cb = (cb0, cb1)
    ob = (ob0, ob1)
    sx = (sx0, sx1)
    sc = (sc0, sc1)
    so = (so0, so1)

    def start_in(bi, off):
        pltpu.async_copy(x_hbm.at[pl.ds(off, chunk)], xb[bi], sx[bi])
        pltpu.async_copy(cid_hbm.at[pl.ds(off, chunk)], cb[bi], sc[bi])

    def wait_in(bi):
        pltpu.make_async_copy(x_hbm.at[pl.ds(0, chunk)], xb[bi], sx[bi]).wait()
        pltpu.make_async_copy(cid_hbm.at[pl.ds(0, chunk)], cb[bi],
                              sc[bi]).wait()

    def start_out(bi, off):
        pltpu.async_copy(ob[bi], out_hbm.at[pl.ds(2 * off, 2 * chunk)],
                         so[bi])

    def wait_out(bi):
        pltpu.make_async_copy(ob[bi], out_hbm.at[pl.ds(0, 2 * chunk)],
                              so[bi]).wait()

    def compute(bi):
        x_v, cid_v, out_v = xb[bi], cb[bi], ob[bi]

        @plsc.parallel_loop(0, chunk // 128, unroll=4)
        def blk(bk):
            for s in range(128 // _LANES):
                o = bk * 128 + s * _LANES
                cid = cid_v[pl.ds(o, _LANES)]
                x1 = plsc.load_gather(x1t_v, [cid])
                inv = plsc.load_gather(invt_v, [cid])
                xv = x_v[pl.ds(o, _LANES)]
                na = (xv - x1) * inv
                p = bk * 256 + s * _LANES
                out_v[pl.ds(p, _LANES)] = na
                out_v[pl.ds(p + 128, _LANES)] = 1.0 - na

    ht0 = pltpu.async_copy(coord_hbm, coord_v, so0)
    ht1 = pltpu.async_copy(conn0_hbm, conn0_v, so1)
    ht2 = pltpu.async_copy(conn1_hbm, conn1_v, st0)

    start_in(0, base)
    start_in(1, base + chunk)

    ht0.wait()
    ht1.wait()
    ht2.wait()
    for t in range(n_cells // _LANES):
        ds = pl.ds(t * _LANES, _LANES)
        n0 = conn0_v[ds]
        n1 = conn1_v[ds]
        x0 = plsc.load_gather(coord_v, [n0])
        x1 = plsc.load_gather(coord_v, [n1])
        x1t_v[ds] = x1
        invt_v[ds] = 1.0 / (x0 - x1)

    n2 = n_chunks // 2

    def pair_body(g, _):
        for b in range(2):
            off = base + (2 * g + b) * chunk
            wait_in(b)

            @pl.when(g > 0)
            def _drain():
                wait_out(b)

            compute(b)
            start_out(b, off)

            @pl.when(g < n2 - 1)
            def _prefetch():
                start_in(b, off + 2 * chunk)

        return _

    lax.fori_loop(0, n2, pair_body, None)
    wait_out(0)
    wait_out(1)


def kernel(x, cell_id, coordinates, connectivity):
    n_pts = x.shape[0]
    n_nodes = coordinates.shape[0]
    n_cells = connectivity.shape[0]
    n_workers = 32
    chunk = 8192

    coord_flat = coordinates[:, 0]
    conn0 = connectivity[:, 0]
    conn1 = connectivity[:, 1]

    mesh = plsc.VectorSubcoreMesh(core_axis_name="c", subcore_axis_name="s")
    body = functools.partial(_tec_kernel, n_pts, n_workers, chunk, n_cells)
    out_flat = pl.kernel(
        body,
        mesh=mesh,
        out_type=jax.ShapeDtypeStruct((2 * n_pts,), jnp.float32),
        compiler_params=pltpu.CompilerParams(needs_layout_passes=False),
        scratch_types=[
            pltpu.VMEM((n_nodes,), jnp.float32),
            pltpu.VMEM((n_cells,), jnp.int32),
            pltpu.VMEM((n_cells,), jnp.int32),
            pltpu.VMEM((n_cells,), jnp.float32),
            pltpu.VMEM((n_cells,), jnp.float32),
            pltpu.VMEM((chunk,), jnp.float32),
            pltpu.VMEM((chunk,), jnp.float32),
            pltpu.VMEM((chunk,), jnp.int32),
            pltpu.VMEM((chunk,), jnp.int32),
            pltpu.VMEM((2 * chunk,), jnp.float32),
            pltpu.VMEM((2 * chunk,), jnp.float32),
            pltpu.SemaphoreType.DMA,
            pltpu.SemaphoreType.DMA,
            pltpu.SemaphoreType.DMA,
            pltpu.SemaphoreType.DMA,
            pltpu.SemaphoreType.DMA,
            pltpu.SemaphoreType.DMA,
            pltpu.SemaphoreType.DMA,
        ],
    )(x, cell_id, coord_flat, conn0, conn1)
    return out_flat.reshape(n_pts // 128, 2, 128).transpose(0, 2, 1).reshape(
        n_pts, 2)
